# baseline (speedup 1.0000x reference)
import functools

import jax
import jax.numpy as jnp
from jax import lax
from jax.experimental import pallas as pl
from jax.experimental.pallas import tpu as pltpu
from jax.experimental.pallas import tpu_sc as plsc
from jax.scipy.linalg import block_diag

F32 = jnp.float32
_NC, _NS = 2, 16
_HI = lax.Precision.HIGHEST


def _stats_body(names_ref, out_ref):
    g = names_ref[...]
    n = g.shape[0] * g.shape[1]
    m = jnp.sum(g) / n
    var = jnp.sum((g - m) ** 2) / n
    out_ref[0, 0] = m
    out_ref[0, 1] = jnp.maximum(jnp.sqrt(var), 1e-6)


def _node_encode_body(stats_ref, names_ref, w1_ref, b1_ref, w2_ref, b2_ref,
                      out0_ref, out1_ref):
    m = stats_ref[0, 0]
    s = stats_ref[0, 1]
    g = (names_ref[...] - m) / s
    hid = jnp.maximum(g * w1_ref[...] + b1_ref[...], 0.0)
    x = jnp.dot(hid, w2_ref[...], preferred_element_type=F32) + b2_ref[...]
    out0_ref[...] = x[:, :16]
    out1_ref[...] = x[:, 16:]


def _eterm_body(a_ref, w1t_ref, b1t_ref, bdw2_ref, b2t_ref, bdwe_ref, bet_ref,
                out_ref):
    a = a_ref[...]
    blk, pack = a.shape
    ab = jnp.broadcast_to(a[:, :, None], (blk, pack, 32)).reshape(blk,
                                                                  pack * 32)
    hid = jnp.maximum(ab * w1t_ref[...] + b1t_ref[...], 0.0)
    ee = jnp.dot(hid, bdw2_ref[...], preferred_element_type=F32) + b2t_ref[...]
    out_ref[...] = (jnp.dot(ee, bdwe_ref[...], preferred_element_type=F32)
                    + bet_ref[...])


def _conv1_mlp_body(x0_ref, x1_ref, p00_ref, p01_ref, p10_ref, p11_ref,
                    w1a_ref, w1b_ref, b1_ref, w2_ref, b2_ref,
                    o0_ref, o1_ref, o2_ref, o3_ref):
    h0 = x0_ref[...] + p00_ref[...] + p01_ref[...]
    h1 = x1_ref[...] + p10_ref[...] + p11_ref[...]
    hid = jnp.maximum(
        jnp.dot(h0, w1a_ref[...], preferred_element_type=F32)
        + jnp.dot(h1, w1b_ref[...], preferred_element_type=F32)
        + b1_ref[...], 0.0)
    x2 = jnp.maximum(
        jnp.dot(hid, w2_ref[...], preferred_element_type=F32) + b2_ref[...], 0.0)
    o0_ref[...] = x2[:, 0:16]
    o1_ref[...] = x2[:, 16:32]
    o2_ref[...] = x2[:, 32:48]
    o3_ref[...] = x2[:, 48:64]


def _conv2_mlp_pool_body(x0_ref, x1_ref, x2_ref, x3_ref,
                         q00_ref, q01_ref, q10_ref, q11_ref,
                         q20_ref, q21_ref, q30_ref, q31_ref,
                         bid_ref, w10_ref, w11_ref, w12_ref, w13_ref,
                         b1_ref, w2_ref, b2_ref, sums_ref, cnts_ref):
    i = pl.program_id(0)
    xq = (x0_ref, x1_ref, x2_ref, x3_ref)
    pq = ((q00_ref, q01_ref), (q10_ref, q11_ref),
          (q20_ref, q21_ref), (q30_ref, q31_ref))
    wq = (w10_ref, w11_ref, w12_ref, w13_ref)
    acc = b1_ref[...]
    for p in range(4):
        hq = xq[p][...] + pq[p][0][...] + pq[p][1][...]
        acc = acc + jnp.dot(hq, wq[p][...], preferred_element_type=F32)
    hid = jnp.maximum(acc, 0.0)
    x3 = jnp.maximum(
        jnp.dot(hid, w2_ref[...], preferred_element_type=F32) + b2_ref[...], 0.0)
    nb = bid_ref.shape[0]
    ngraph = cnts_ref.shape[0]
    onehot = (bid_ref[...] ==
              lax.broadcasted_iota(jnp.int32, (nb, ngraph), 1)).astype(F32)
    dnums = (((0,), (0,)), ((), ()))
    psums = lax.dot_general(onehot, x3, dnums, preferred_element_type=F32,
                            precision=_HI)
    pcnts = lax.dot_general(onehot, jnp.ones((nb, 1), F32), dnums,
                            preferred_element_type=F32, precision=_HI)

    @pl.when(i == 0)
    def _():
        sums_ref[...] = psums
        cnts_ref[...] = pcnts

    @pl.when(i != 0)
    def _():
        sums_ref[...] += psums
        cnts_ref[...] += pcnts


def _classifier_body(sums_ref, cnts_ref, depth_ref, w1a_ref, w1b_ref, b1_ref,
                     w2_ref, b2_ref, out_ref):
    cnt = jnp.maximum(cnts_ref[...], 1.0)
    pooled = sums_ref[...] / cnt
    d = depth_ref[...]
    nb = d.shape[0]
    m = jnp.sum(d) / nb
    s = jnp.sqrt(jnp.sum((d - m) ** 2) / nb) + 1e-6
    dn = (d - m) / s
    h = (jnp.dot(pooled, w1a_ref[...], preferred_element_type=F32)
         + dn * w1b_ref[...] + b1_ref[...])
    h = jnp.maximum(h, 0.0)
    out_ref[...] = jnp.dot(h, w2_ref[...], preferred_element_type=F32) + b2_ref[...]


_CR = 5
_NBUF = 4


def _sc_segment_agg(tables, edata, ev, pack, npad):
    num_p = len(tables)
    rows = edata.shape[0]
    rows_per_core = rows // _NC
    rows_per_tile = rows_per_core // _NS
    n_chunks = rows_per_tile // _CR
    n_quads = n_chunks // _NBUF
    tile_nrows = npad // _NS
    zchunks = tile_nrows // 128
    rpp = 128 // pack

    mesh = plsc.VectorSubcoreMesh(core_axis_name="c", subcore_axis_name="s")

    scratch = [pltpu.VMEM((_CR, 2, 128), jnp.int32) for _ in range(_NBUF)]
    scratch += [pltpu.VMEM((_CR * rpp, pack, 16), F32) for _ in range(2)]
    scratch += [pltpu.VMEM((_CR, 128, 16), F32) for _ in range(_NBUF)]
    scratch += [
        pltpu.VMEM((128, 16), F32),
        pltpu.VMEM_SHARED((npad, 16), F32),
    ]
    scratch += [pltpu.SemaphoreType.DMA for _ in range(2 * _NBUF + 1)]

    @functools.partial(
        pl.kernel,
        out_type=tuple(jax.ShapeDtypeStruct((2 * npad, 16), F32)
                       for _ in range(num_p)),
        mesh=mesh,
        compiler_params=pltpu.CompilerParams(use_tc_tiling_on_sc=False),
        scratch_types=scratch)
    def k(*refs):
        table_hs = refs[:num_p]
        edata_h, ev_h = refs[num_p:num_p + 2]
        outs = refs[num_p + 2:num_p + 2 + num_p]
        sc = refs[num_p + 2 + num_p:]
        ed_bufs = sc[:_NBUF]
        ebufs = sc[_NBUF:_NBUF + 2]
        rows_bufs = sc[_NBUF + 2:2 * _NBUF + 2]
        zero_v, agg_s = sc[2 * _NBUF + 2:2 * _NBUF + 4]
        sems = sc[2 * _NBUF + 4:]
        gsems = sems[:_NBUF]
        ssems = sems[_NBUF:2 * _NBUF]
        zsem = sems[2 * _NBUF]
        cid = lax.axis_index("c")
        sid = lax.axis_index("s")

        def zrow(r, carry):
            zero_v[r, :] = jnp.zeros((16,), F32)
            return carry
        lax.fori_loop(0, 128, zrow, 0)
        zbase = sid * tile_nrows
        base = cid * rows_per_core + sid * rows_per_tile

        def drain(semref, dst_ref, count, src_h):
            def dr(i, carry):
                pltpu.make_async_copy(src_h.at[pl.ds(0, 128)], dst_ref,
                                      semref).wait()
                return carry
            lax.fori_loop(0, count, dr, 0)

        for p in range(num_p):
            table_h = table_hs[p]

            def zissue(zi, carry):
                pltpu.async_copy(zero_v,
                                 agg_s.at[pl.ds(zbase + zi * 128, 128)], zsem)
                return carry
            lax.fori_loop(0, zchunks, zissue, 0)
            drain(zsem, zero_v, zchunks, table_h)
            plsc.subcore_barrier()

            def issue_chunk(c, b):
                r0 = base + c * _CR
                pltpu.sync_copy(edata_h.at[pl.ds(r0, _CR)], ed_bufs[b])
                pltpu.sync_copy(
                    ev_h.at[pl.ds(r0 * rpp, _CR * rpp), :,
                            pl.ds(p * 16, 16)], ebufs[b % 2])
                for j in range(_CR):
                    pltpu.async_copy(table_h.at[ed_bufs[b].at[j, 0]],
                                     rows_bufs[b].at[j], gsems[b])

            issue_chunk(0, 0)
            issue_chunk(1, 1)

            def quad(t, carry):
                for kk in range(_NBUF):
                    c = t * _NBUF + kk
                    rows_k = rows_bufs[kk]
                    ed_k = ed_bufs[kk]
                    eb_k = ebufs[kk % 2]
                    w = (kk + 2) % _NBUF
                    drain(gsems[kk], rows_k.at[0], _CR, table_h)

                    def jbody(j, jcarry):
                        @plsc.parallel_loop(0, rpp)
                        def _(rr):
                            for u in range(pack):
                                r = rr * pack + u
                                v = (rows_k[j, r, :]
                                     + eb_k[j * rpp + rr, u, :])
                                rows_k[j, r, :] = jnp.maximum(v, 0.0)
                        return jcarry
                    lax.fori_loop(0, _CR, jbody, 0)
                    for j in range(_CR):
                        pltpu.async_copy(rows_k.at[j],
                                         agg_s.at[ed_k.at[j, 1]],
                                         ssems[kk], add=True)
                    c2 = c + 2

                    @pl.when(c2 < n_chunks)
                    def _():
                        @pl.when(c >= 2)
                        def _():
                            drain(ssems[w], rows_bufs[w].at[0], _CR, table_h)
                        issue_chunk(c2, w)
                return carry
            lax.fori_loop(0, n_quads, quad, 0)
            for b in range(_NBUF):
                drain(ssems[b], rows_bufs[b].at[0], _CR, table_h)
            plsc.subcore_barrier()

            pltpu.sync_copy(agg_s.at[pl.ds(zbase, tile_nrows)],
                            outs[p].at[pl.ds(cid * npad + zbase, tile_nrows)])
            plsc.subcore_barrier()

    return k(*tables, edata, ev)


def kernel(node_names, edge_index, edge_attr, batch_ids, depth,
           id_W1, id_b1, id_W2, id_b2,
           ed_W1, ed_b1, ed_W2, ed_b2,
           c1_We, c1_be, c1_W1, c1_b1, c1_W2, c1_b2,
           c2_We, c2_be, c2_W1, c2_b1, c2_W2, c2_b2,
           cl_W1, cl_b1, cl_W2, cl_b2):
    n = node_names.shape[0]
    e = edge_index.shape[1]
    bg = depth.shape[0]
    npad = -(-n // 2048) * 2048
    epad = -(-e // (32 * 8 * 128)) * (32 * 8 * 128)
    row_blk = npad // 16

    raw = node_names.astype(F32)
    names2d = raw.reshape(n // 8, 8)
    names_pad = jnp.pad(raw, (0, npad - n)).reshape(npad, 1)
    trash = n + (jnp.arange(epad - e, dtype=jnp.int32) % 1024)
    src_p = jnp.pad(edge_index[0], (0, epad - e))
    dst_p = jnp.concatenate([edge_index[1], trash])
    edata = jnp.stack([src_p.reshape(epad // 128, 128),
                       dst_p.reshape(epad // 128, 128)], axis=1)
    a_pad = jnp.pad(edge_attr[:, 0], (0, epad - e))
    bid_pad = jnp.pad(batch_ids, (0, npad - n),
                      constant_values=bg).reshape(npad, 1)

    w1row = ed_W1[0]
    w1t4 = jnp.tile(w1row, 4).reshape(1, 128)
    b1t4 = jnp.tile(ed_b1, 4).reshape(1, 128)
    bd4_w2 = block_diag(ed_W2, ed_W2, ed_W2, ed_W2)
    b2t4 = jnp.tile(ed_b2, 4).reshape(1, 128)
    bd4_we = block_diag(c1_We, c1_We, c1_We, c1_We)
    bet4 = jnp.tile(c1_be, 4).reshape(1, 128)
    w1t2 = jnp.tile(w1row, 2).reshape(1, 64)
    b1t2 = jnp.tile(ed_b1, 2).reshape(1, 64)
    bd2_w2 = block_diag(ed_W2, ed_W2)
    b2t2 = jnp.tile(ed_b2, 2).reshape(1, 64)
    bd2_we = block_diag(c2_We, c2_We)
    bet2 = jnp.tile(c2_be, 2).reshape(1, 128)

    full2d = lambda shp: pl.BlockSpec(shp, lambda i: (0, 0))
    rowspec = pl.BlockSpec((row_blk, 16), lambda i: (i, 0))

    stats = pl.pallas_call(
        _stats_body,
        out_shape=jax.ShapeDtypeStruct((1, 2), F32),
        in_specs=[pl.BlockSpec((n // 8, 8), lambda: (0, 0))],
        out_specs=pl.BlockSpec(memory_space=pltpu.SMEM),
    )(names2d)

    x0, x1 = pl.pallas_call(
        _node_encode_body,
        grid=(16,),
        out_shape=(jax.ShapeDtypeStruct((npad, 16), F32),
                   jax.ShapeDtypeStruct((npad, 16), F32)),
        in_specs=[
            pl.BlockSpec(memory_space=pltpu.SMEM),
            pl.BlockSpec((row_blk, 1), lambda i: (i, 0)),
            full2d((1, 32)), full2d((1, 32)), full2d((32, 32)), full2d((1, 32)),
        ],
        out_specs=(rowspec, rowspec),
    )(stats, names_pad, id_W1, id_b1.reshape(1, 32), id_W2,
      id_b2.reshape(1, 32))

    ep4, ep2 = epad // 4, epad // 2
    e1p = pl.pallas_call(
        _eterm_body,
        grid=(32,),
        out_shape=jax.ShapeDtypeStruct((ep4, 128), F32),
        in_specs=[
            pl.BlockSpec((ep4 // 32, 4), lambda i: (i, 0)),
            full2d((1, 128)), full2d((1, 128)), full2d((128, 128)),
            full2d((1, 128)), full2d((128, 128)), full2d((1, 128)),
        ],
        out_specs=pl.BlockSpec((ep4 // 32, 128), lambda i: (i, 0)),
    )(a_pad.reshape(ep4, 4), w1t4, b1t4, bd4_w2, b2t4, bd4_we, bet4)
    e2p = pl.pallas_call(
        _eterm_body,
        grid=(64,),
        out_shape=jax.ShapeDtypeStruct((ep2, 128), F32),
        in_specs=[
            pl.BlockSpec((ep2 // 64, 2), lambda i: (i, 0)),
            full2d((1, 64)), full2d((1, 64)), full2d((64, 64)),
            full2d((1, 64)), full2d((64, 128)), full2d((1, 128)),
        ],
        out_specs=pl.BlockSpec((ep2 // 64, 128), lambda i: (i, 0)),
    )(a_pad.reshape(ep2, 2), w1t2, b1t2, bd2_w2, b2t2, bd2_we, bet2)

    agg1 = _sc_segment_agg([x0, x1], edata, e1p.reshape(ep4, 4, 32), 4, npad)
    p00, p01 = agg1[0][:npad], agg1[0][npad:]
    p10, p11 = agg1[1][:npad], agg1[1][npad:]

    x2q = pl.pallas_call(
        _conv1_mlp_body,
        grid=(16,),
        out_shape=tuple(jax.ShapeDtypeStruct((npad, 16), F32)
                        for _ in range(4)),
        in_specs=[
            rowspec, rowspec, rowspec, rowspec, rowspec, rowspec,
            full2d((16, 64)), full2d((16, 64)), full2d((1, 64)),
            full2d((64, 64)), full2d((1, 64)),
        ],
        out_specs=(rowspec, rowspec, rowspec, rowspec),
    )(x0, x1, p00, p01, p10, p11, c1_W1[:16], c1_W1[16:],
      c1_b1.reshape(1, 64), c1_W2, c1_b2.reshape(1, 64))

    agg2 = _sc_segment_agg(list(x2q), edata, e2p.reshape(ep2, 2, 64), 2, npad)
    qparts = []
    for arr in agg2:
        qparts.extend([arr[:npad], arr[npad:]])

    sums, cnts = pl.pallas_call(
        _conv2_mlp_pool_body,
        grid=(16,),
        out_shape=(jax.ShapeDtypeStruct((bg, 64), F32),
                   jax.ShapeDtypeStruct((bg, 1), F32)),
        in_specs=[
            rowspec, rowspec, rowspec, rowspec,
            rowspec, rowspec, rowspec, rowspec,
            rowspec, rowspec, rowspec, rowspec,
            pl.BlockSpec((row_blk, 1), lambda i: (i, 0)),
            full2d((16, 64)), full2d((16, 64)), full2d((16, 64)),
            full2d((16, 64)),
            full2d((1, 64)), full2d((64, 64)), full2d((1, 64)),
        ],
        out_specs=(pl.BlockSpec((bg, 64), lambda i: (0, 0)),
                   pl.BlockSpec((bg, 1), lambda i: (0, 0))),
    )(*x2q, *qparts, bid_pad,
      c2_W1[0:16], c2_W1[16:32], c2_W1[32:48], c2_W1[48:64],
      c2_b1.reshape(1, 64), c2_W2, c2_b2.reshape(1, 64))

    logits = pl.pallas_call(
        _classifier_body,
        out_shape=jax.ShapeDtypeStruct((bg, 1), F32),
        in_specs=[
            pl.BlockSpec((bg, 64), lambda: (0, 0)),
            pl.BlockSpec((bg, 1), lambda: (0, 0)),
            pl.BlockSpec((bg, 1), lambda: (0, 0)),
            pl.BlockSpec((64, 64), lambda: (0, 0)),
            pl.BlockSpec((1, 64), lambda: (0, 0)),
            pl.BlockSpec((1, 64), lambda: (0, 0)),
            pl.BlockSpec((64, 1), lambda: (0, 0)),
            pl.BlockSpec((1, 1), lambda: (0, 0)),
        ],
    )(sums, cnts, depth.reshape(bg, 1), cl_W1[:64], cl_W1[64:],
      cl_b1.reshape(1, 64), cl_W2, cl_b2.reshape(1, 1))

    return logits[:, 0]
